# baseline (speedup 1.0000x reference)
import functools

import jax
import jax.numpy as jnp
from jax import lax
from jax.experimental import pallas as pl
from jax.experimental.pallas import tpu as pltpu
from jax.experimental.pallas import tpu_sc as plsc

N_EXPERTS = 8
D_MODEL = 1024
D_FF = 2048
ALPHA = 1.702
BETA = 1.0
LIMIT = 7.0
T_TOKENS = 2048
TOP_K = 2

BM = 256
NBLK = (T_TOKENS * TOP_K) // BM + N_EXPERTS
NPAD = NBLK * BM

NW = 32
TPW = T_TOKENS // NW
CH = 32


def _router_body(x_ref, gw_ref, gb_ref, slots_ref, wts_ref, counts_ref):
    x = x_ref[...]
    logits = jax.lax.dot_general(
        x, gw_ref[...], (((1,), (1,)), ((), ())),
        preferred_element_type=jnp.float32) + gb_ref[...]

    lane = jax.lax.broadcasted_iota(jnp.int32, (T_TOKENS, N_EXPERTS), 1)
    m1 = jnp.max(logits, axis=1, keepdims=True)
    i1 = jnp.min(jnp.where(logits == m1, lane, N_EXPERTS), axis=1,
                 keepdims=True)
    masked = jnp.where(lane == i1, -jnp.inf, logits)
    m2 = jnp.max(masked, axis=1, keepdims=True)
    i2 = jnp.min(jnp.where(masked == m2, lane, N_EXPERTS), axis=1,
                 keepdims=True)
    w1 = jax.nn.sigmoid(m1 - m2)
    w2 = 1.0 - w1

    oh1 = (lane == i1).astype(jnp.float32)
    oh2 = (lane == i2).astype(jnp.float32)
    oh = oh1 + oh2

    r = jax.lax.broadcasted_iota(jnp.int32, (T_TOKENS, T_TOKENS), 0)
    c = jax.lax.broadcasted_iota(jnp.int32, (T_TOKENS, T_TOKENS), 1)
    tril = (c < r).astype(jnp.float32)
    cnt_before = jax.lax.dot_general(
        tril, oh, (((1,), (0,)), ((), ())),
        preferred_element_type=jnp.float32)
    counts = jnp.sum(oh, axis=0, keepdims=True)

    aligned = jnp.ceil(counts / BM) * BM
    e_r = jax.lax.broadcasted_iota(jnp.int32, (N_EXPERTS, N_EXPERTS), 0)
    e_c = jax.lax.broadcasted_iota(jnp.int32, (N_EXPERTS, N_EXPERTS), 1)
    tril8 = (e_r < e_c).astype(jnp.float32)
    start = jax.lax.dot_general(
        aligned, tril8, (((1,), (0,)), ((), ())),
        preferred_element_type=jnp.float32)

    rank1 = jnp.sum(oh1 * cnt_before, axis=1, keepdims=True)
    rank2 = jnp.sum(oh2 * cnt_before, axis=1, keepdims=True)
    base1 = jnp.sum(oh1 * start, axis=1, keepdims=True)
    base2 = jnp.sum(oh2 * start, axis=1, keepdims=True)
    slot1 = (base1 + rank1).astype(jnp.int32)
    slot2 = (base2 + rank2).astype(jnp.int32)

    slots_ref[...] = jnp.concatenate([slot1, slot2], axis=1)
    wts_ref[...] = jnp.concatenate([w1, w2], axis=1)
    counts_ref[...] = counts


def _run_router(x, gate_w, gate_b):
    return pl.pallas_call(
        _router_body,
        out_shape=(
            jax.ShapeDtypeStruct((T_TOKENS, 2), jnp.int32),
            jax.ShapeDtypeStruct((T_TOKENS, 2), jnp.float32),
            jax.ShapeDtypeStruct((1, N_EXPERTS), jnp.float32),
        ),
    )(x, gate_w, gate_b.reshape(1, N_EXPERTS))


def _make_dispatch():
    mesh = plsc.VectorSubcoreMesh(core_axis_name="c", subcore_axis_name="s")

    @functools.partial(
        pl.kernel, mesh=mesh,
        out_type=jax.ShapeDtypeStruct((NPAD, D_MODEL), jnp.float32),
        scratch_types=[
            pltpu.VMEM((2, TPW), jnp.int32),
            pltpu.VMEM((TPW, D_MODEL), jnp.float32),
            pltpu.SemaphoreType.DMA,
        ],
    )
    def dispatch(x_hbm, slots_hbm, xs_hbm, idx_v, rows_v, sem):
        wid = lax.axis_index("s") * 2 + lax.axis_index("c")
        base = wid * TPW
        pltpu.sync_copy(slots_hbm.at[wid], idx_v)
        pltpu.sync_copy(x_hbm.at[pl.ds(base, TPW)], rows_v)
        pltpu.async_copy(rows_v, xs_hbm.at[idx_v.at[0]], sem).wait()
        pltpu.async_copy(rows_v, xs_hbm.at[idx_v.at[1]], sem).wait()

    return dispatch


def _grouped_body(be_ref, act_ref, xs_ref, wglu_ref, wlin_ref, bglu_ref,
                  blin_ref, wd_ref, bd_ref, ys_ref):
    b = pl.program_id(0)

    @pl.when(act_ref[b] == 1)
    def _():
        xb = xs_ref[...].astype(jnp.bfloat16)
        h_glu = jax.lax.dot_general(
            xb, wglu_ref[...].astype(jnp.bfloat16),
            (((1,), (0,)), ((), ())),
            preferred_element_type=jnp.float32) + bglu_ref[...]
        h_lin = jax.lax.dot_general(
            xb, wlin_ref[...].astype(jnp.bfloat16),
            (((1,), (0,)), ((), ())),
            preferred_element_type=jnp.float32) + blin_ref[...]
        h_glu = jnp.minimum(h_glu, LIMIT)
        h_lin = jnp.clip(h_lin, -LIMIT, LIMIT)
        act = h_glu * jax.nn.sigmoid(ALPHA * h_glu) * (h_lin + BETA)
        y = jax.lax.dot_general(
            act.astype(jnp.bfloat16), wd_ref[...].astype(jnp.bfloat16),
            (((1,), (0,)), ((), ())),
            preferred_element_type=jnp.float32)
        ys_ref[...] = y + bd_ref[...]


def _run_grouped(xs, w_gate_up, bgu3, w_down, bd3, block_expert, active):
    grid_spec = pltpu.PrefetchScalarGridSpec(
        num_scalar_prefetch=2,
        grid=(NBLK,),
        in_specs=[
            pl.BlockSpec((BM, D_MODEL), lambda b, be, act: (b, 0)),
            pl.BlockSpec((None, D_MODEL, D_FF),
                         lambda b, be, act: (be[b], 0, 0)),
            pl.BlockSpec((None, D_MODEL, D_FF),
                         lambda b, be, act: (be[b], 0, 1)),
            pl.BlockSpec((None, 1, D_FF), lambda b, be, act: (be[b], 0, 0)),
            pl.BlockSpec((None, 1, D_FF), lambda b, be, act: (be[b], 0, 1)),
            pl.BlockSpec((None, D_FF, D_MODEL),
                         lambda b, be, act: (be[b], 0, 0)),
            pl.BlockSpec((None, 1, D_MODEL),
                         lambda b, be, act: (be[b], 0, 0)),
        ],
        out_specs=pl.BlockSpec((BM, D_MODEL), lambda b, be, act: (b, 0)),
    )
    return pl.pallas_call(
        _grouped_body,
        grid_spec=grid_spec,
        out_shape=jax.ShapeDtypeStruct((NPAD, D_MODEL), jnp.float32),
    )(block_expert, active, xs, w_gate_up, w_gate_up, bgu3, bgu3,
      w_down, bd3)


def _make_combine():
    mesh = plsc.VectorSubcoreMesh(core_axis_name="c", subcore_axis_name="s")

    @functools.partial(
        pl.kernel, mesh=mesh,
        out_type=jax.ShapeDtypeStruct((T_TOKENS, D_MODEL), jnp.float32),
        scratch_types=[
            pltpu.VMEM((2, TPW), jnp.int32),
            pltpu.VMEM((2, TPW, 16), jnp.float32),
            pltpu.VMEM((CH, D_MODEL), jnp.float32),
            pltpu.VMEM((CH, D_MODEL), jnp.float32),
            pltpu.VMEM((CH, D_MODEL), jnp.float32),
            pltpu.SemaphoreType.DMA,
        ],
    )
    def combine(ys_hbm, slots_hbm, wts_hbm, out_hbm, idx_v, w_v, r0, r1, o,
                sem):
        wid = lax.axis_index("s") * 2 + lax.axis_index("c")
        base = wid * TPW
        pltpu.sync_copy(slots_hbm.at[wid], idx_v)
        pltpu.sync_copy(wts_hbm.at[wid], w_v)
        for ci in range(TPW // CH):
            pltpu.async_copy(
                ys_hbm.at[idx_v.at[0, pl.ds(ci * CH, CH)]], r0, sem).wait()
            pltpu.async_copy(
                ys_hbm.at[idx_v.at[1, pl.ds(ci * CH, CH)]], r1, sem).wait()
            for j in range(CH):
                w0 = w_v[0, ci * CH + j, :]
                w1 = w_v[1, ci * CH + j, :]

                def chunk_add(k, _, j=j, w0=w0, w1=w1):
                    sl = pl.ds(k * 16, 16)
                    o[j, sl] = w0 * r0[j, sl] + w1 * r1[j, sl]
                    return 0
                lax.fori_loop(0, D_MODEL // 16, chunk_add, 0)
            pltpu.sync_copy(o, out_hbm.at[pl.ds(base + ci * CH, CH)])

    return combine


@jax.jit
def kernel(x, gate_w, gate_b, w_gate_up, b_gate_up, w_down, b_down):
    slots, wts, counts = _run_router(x, gate_w, gate_b)

    nb = jnp.ceil(counts[0] / BM).astype(jnp.int32)
    cnb = jnp.cumsum(nb)
    total_nb = cnb[-1]
    barange = jnp.arange(NBLK, dtype=jnp.int32)
    be_raw = jnp.minimum(
        (barange[:, None] >= cnb[None, :]).sum(axis=1), N_EXPERTS - 1
    ).astype(jnp.int32)
    active = (barange < total_nb).astype(jnp.int32)
    last_e = be_raw[jnp.maximum(total_nb - 1, 0)]
    block_expert = jnp.where(active == 1, be_raw, last_e)

    slots_w = slots.reshape(NW, TPW, 2).transpose(0, 2, 1)
    wts_w = wts.reshape(NW, TPW, 2).transpose(0, 2, 1)
    wts_wb = jnp.broadcast_to(wts_w[..., None],
                              (NW, 2, TPW, 16)) + jnp.zeros(
        (NW, 2, TPW, 16), jnp.float32)

    xs = _make_dispatch()(x, slots_w)

    bgu3 = b_gate_up.reshape(N_EXPERTS, 1, 2 * D_FF)
    bd3 = b_down.reshape(N_EXPERTS, 1, D_MODEL)
    ys = _run_grouped(xs, w_gate_up, bgu3, w_down, bd3, block_expert, active)

    out = _make_combine()(ys, slots_w, wts_wb)
    return out


# speedup vs baseline: 1.0078x; 1.0078x over previous
import functools

import jax
import jax.numpy as jnp
from jax import lax
from jax.experimental import pallas as pl
from jax.experimental.pallas import tpu as pltpu
from jax.experimental.pallas import tpu_sc as plsc

N_EXPERTS = 8
D_MODEL = 1024
D_FF = 2048
ALPHA = 1.702
BETA = 1.0
LIMIT = 7.0
T_TOKENS = 2048
TOP_K = 2

BM = 256
NBLK = (T_TOKENS * TOP_K) // BM + N_EXPERTS
NPAD = NBLK * BM

NW = 32
TPW = T_TOKENS // NW
CH = 32


def _router_body(x_ref, gw_ref, gb_ref, slots_ref, wts_ref, counts_ref):
    x = x_ref[...]
    logits = jax.lax.dot_general(
        x, gw_ref[...], (((1,), (1,)), ((), ())),
        preferred_element_type=jnp.float32) + gb_ref[...]

    lane = jax.lax.broadcasted_iota(jnp.int32, (T_TOKENS, N_EXPERTS), 1)
    m1 = jnp.max(logits, axis=1, keepdims=True)
    i1 = jnp.min(jnp.where(logits == m1, lane, N_EXPERTS), axis=1,
                 keepdims=True)
    masked = jnp.where(lane == i1, -jnp.inf, logits)
    m2 = jnp.max(masked, axis=1, keepdims=True)
    i2 = jnp.min(jnp.where(masked == m2, lane, N_EXPERTS), axis=1,
                 keepdims=True)
    w1 = jax.nn.sigmoid(m1 - m2)
    w2 = 1.0 - w1

    oh1 = (lane == i1).astype(jnp.float32)
    oh2 = (lane == i2).astype(jnp.float32)
    oh = oh1 + oh2

    r = jax.lax.broadcasted_iota(jnp.int32, (T_TOKENS, T_TOKENS), 0)
    c = jax.lax.broadcasted_iota(jnp.int32, (T_TOKENS, T_TOKENS), 1)
    tril = (c < r).astype(jnp.float32)
    cnt_before = jax.lax.dot_general(
        tril, oh, (((1,), (0,)), ((), ())),
        preferred_element_type=jnp.float32)
    counts = jnp.sum(oh, axis=0, keepdims=True)

    aligned = jnp.ceil(counts / BM) * BM
    e_r = jax.lax.broadcasted_iota(jnp.int32, (N_EXPERTS, N_EXPERTS), 0)
    e_c = jax.lax.broadcasted_iota(jnp.int32, (N_EXPERTS, N_EXPERTS), 1)
    tril8 = (e_r < e_c).astype(jnp.float32)
    start = jax.lax.dot_general(
        aligned, tril8, (((1,), (0,)), ((), ())),
        preferred_element_type=jnp.float32)

    rank1 = jnp.sum(oh1 * cnt_before, axis=1, keepdims=True)
    rank2 = jnp.sum(oh2 * cnt_before, axis=1, keepdims=True)
    base1 = jnp.sum(oh1 * start, axis=1, keepdims=True)
    base2 = jnp.sum(oh2 * start, axis=1, keepdims=True)
    slot1 = (base1 + rank1).astype(jnp.int32)
    slot2 = (base2 + rank2).astype(jnp.int32)

    slots_ref[...] = jnp.concatenate([slot1, slot2], axis=1)
    wts_ref[...] = jnp.concatenate([w1, w2], axis=1)
    counts_ref[...] = counts


def _run_router(x, gate_w, gate_b):
    return pl.pallas_call(
        _router_body,
        out_shape=(
            jax.ShapeDtypeStruct((T_TOKENS, 2), jnp.int32),
            jax.ShapeDtypeStruct((T_TOKENS, 2), jnp.float32),
            jax.ShapeDtypeStruct((1, N_EXPERTS), jnp.float32),
        ),
    )(x, gate_w, gate_b.reshape(1, N_EXPERTS))


def _make_dispatch():
    mesh = plsc.VectorSubcoreMesh(core_axis_name="c", subcore_axis_name="s")

    @functools.partial(
        pl.kernel, mesh=mesh,
        out_type=jax.ShapeDtypeStruct((NPAD, D_MODEL), jnp.float32),
        scratch_types=[
            pltpu.VMEM((2, TPW), jnp.int32),
            pltpu.VMEM((TPW, D_MODEL), jnp.float32),
            pltpu.SemaphoreType.DMA,
        ],
    )
    def dispatch(x_hbm, slots_hbm, xs_hbm, idx_v, rows_v, sem):
        wid = lax.axis_index("s") * 2 + lax.axis_index("c")
        base = wid * TPW
        pltpu.sync_copy(slots_hbm.at[wid], idx_v)
        pltpu.sync_copy(x_hbm.at[pl.ds(base, TPW)], rows_v)
        pltpu.async_copy(rows_v, xs_hbm.at[idx_v.at[0]], sem).wait()
        pltpu.async_copy(rows_v, xs_hbm.at[idx_v.at[1]], sem).wait()

    return dispatch


def _grouped_body(be_ref, act_ref, xs_ref, wglu_ref, wlin_ref, bglu_ref,
                  blin_ref, wd_ref, bd_ref, ys_ref):
    b = pl.program_id(0)

    @pl.when(act_ref[b] == 1)
    def _():
        xb = xs_ref[...]
        h_glu = jax.lax.dot_general(
            xb, wglu_ref[...], (((1,), (0,)), ((), ())),
            preferred_element_type=jnp.float32) + bglu_ref[...]
        h_lin = jax.lax.dot_general(
            xb, wlin_ref[...], (((1,), (0,)), ((), ())),
            preferred_element_type=jnp.float32) + blin_ref[...]
        h_glu = jnp.minimum(h_glu, LIMIT)
        h_lin = jnp.clip(h_lin, -LIMIT, LIMIT)
        act = h_glu * jax.nn.sigmoid(ALPHA * h_glu) * (h_lin + BETA)
        y = jax.lax.dot_general(
            act, wd_ref[...], (((1,), (0,)), ((), ())),
            preferred_element_type=jnp.float32)
        ys_ref[...] = y + bd_ref[...]


def _run_grouped(xs, w_gate_up, bgu3, w_down, bd3, block_expert, active):
    grid_spec = pltpu.PrefetchScalarGridSpec(
        num_scalar_prefetch=2,
        grid=(NBLK,),
        in_specs=[
            pl.BlockSpec((BM, D_MODEL), lambda b, be, act: (b, 0)),
            pl.BlockSpec((None, D_MODEL, D_FF),
                         lambda b, be, act: (be[b], 0, 0)),
            pl.BlockSpec((None, D_MODEL, D_FF),
                         lambda b, be, act: (be[b], 0, 1)),
            pl.BlockSpec((None, 1, D_FF), lambda b, be, act: (be[b], 0, 0)),
            pl.BlockSpec((None, 1, D_FF), lambda b, be, act: (be[b], 0, 1)),
            pl.BlockSpec((None, D_FF, D_MODEL),
                         lambda b, be, act: (be[b], 0, 0)),
            pl.BlockSpec((None, 1, D_MODEL),
                         lambda b, be, act: (be[b], 0, 0)),
        ],
        out_specs=pl.BlockSpec((BM, D_MODEL), lambda b, be, act: (b, 0)),
    )
    return pl.pallas_call(
        _grouped_body,
        grid_spec=grid_spec,
        out_shape=jax.ShapeDtypeStruct((NPAD, D_MODEL), jnp.float32),
    )(block_expert, active, xs, w_gate_up, w_gate_up, bgu3, bgu3,
      w_down, bd3)


def _make_combine():
    mesh = plsc.VectorSubcoreMesh(core_axis_name="c", subcore_axis_name="s")

    @functools.partial(
        pl.kernel, mesh=mesh,
        out_type=jax.ShapeDtypeStruct((T_TOKENS, D_MODEL), jnp.float32),
        scratch_types=[
            pltpu.VMEM((2, TPW), jnp.int32),
            pltpu.VMEM((2, TPW, 16), jnp.float32),
            pltpu.VMEM((CH, D_MODEL), jnp.float32),
            pltpu.VMEM((CH, D_MODEL), jnp.float32),
            pltpu.VMEM((CH, D_MODEL), jnp.float32),
            pltpu.SemaphoreType.DMA,
        ],
    )
    def combine(ys_hbm, slots_hbm, wts_hbm, out_hbm, idx_v, w_v, r0, r1, o,
                sem):
        wid = lax.axis_index("s") * 2 + lax.axis_index("c")
        base = wid * TPW
        pltpu.sync_copy(slots_hbm.at[wid], idx_v)
        pltpu.sync_copy(wts_hbm.at[wid], w_v)
        for ci in range(TPW // CH):
            pltpu.async_copy(
                ys_hbm.at[idx_v.at[0, pl.ds(ci * CH, CH)]], r0, sem).wait()
            pltpu.async_copy(
                ys_hbm.at[idx_v.at[1, pl.ds(ci * CH, CH)]], r1, sem).wait()
            for j in range(CH):
                w0 = w_v[0, ci * CH + j, :]
                w1 = w_v[1, ci * CH + j, :]

                def chunk_add(k, _, j=j, w0=w0, w1=w1):
                    sl = pl.ds(k * 16, 16)
                    o[j, sl] = w0 * r0[j, sl] + w1 * r1[j, sl]
                    return 0
                lax.fori_loop(0, D_MODEL // 16, chunk_add, 0)
            pltpu.sync_copy(o, out_hbm.at[pl.ds(base + ci * CH, CH)])

    return combine


@jax.jit
def kernel(x, gate_w, gate_b, w_gate_up, b_gate_up, w_down, b_down):
    slots, wts, counts = _run_router(x, gate_w, gate_b)

    nb = jnp.ceil(counts[0] / BM).astype(jnp.int32)
    cnb = jnp.cumsum(nb)
    total_nb = cnb[-1]
    barange = jnp.arange(NBLK, dtype=jnp.int32)
    be_raw = jnp.minimum(
        (barange[:, None] >= cnb[None, :]).sum(axis=1), N_EXPERTS - 1
    ).astype(jnp.int32)
    active = (barange < total_nb).astype(jnp.int32)
    last_e = be_raw[jnp.maximum(total_nb - 1, 0)]
    block_expert = jnp.where(active == 1, be_raw, last_e)

    slots_w = slots.reshape(NW, TPW, 2).transpose(0, 2, 1)
    wts_w = wts.reshape(NW, TPW, 2).transpose(0, 2, 1)
    wts_wb = jnp.broadcast_to(wts_w[..., None],
                              (NW, 2, TPW, 16)) + jnp.zeros(
        (NW, 2, TPW, 16), jnp.float32)

    xs = _make_dispatch()(x, slots_w)

    bgu3 = b_gate_up.reshape(N_EXPERTS, 1, 2 * D_FF)
    bd3 = b_down.reshape(N_EXPERTS, 1, D_MODEL)
    ys = _run_grouped(xs, w_gate_up, bgu3, w_down, bd3, block_expert, active)

    out = _make_combine()(ys, slots_w, wts_wb)
    return out


# speedup vs baseline: 1.0432x; 1.0351x over previous
import functools

import jax
import jax.numpy as jnp
from jax import lax
from jax.experimental import pallas as pl
from jax.experimental.pallas import tpu as pltpu
from jax.experimental.pallas import tpu_sc as plsc

N_EXPERTS = 8
D_MODEL = 1024
D_FF = 2048
ALPHA = 1.702
BETA = 1.0
LIMIT = 7.0
T_TOKENS = 2048
TOP_K = 2

BM = 256
NBLK = (T_TOKENS * TOP_K) // BM + N_EXPERTS
NPAD = NBLK * BM

NW = 32
TPW = T_TOKENS // NW
CH = 16


def _router_body(x_ref, gw_ref, gb_ref, slots_ref, wts_ref, counts_ref):
    x = x_ref[...]
    logits = jax.lax.dot_general(
        x, gw_ref[...], (((1,), (1,)), ((), ())),
        preferred_element_type=jnp.float32) + gb_ref[...]

    lane = jax.lax.broadcasted_iota(jnp.int32, (T_TOKENS, N_EXPERTS), 1)
    m1 = jnp.max(logits, axis=1, keepdims=True)
    i1 = jnp.min(jnp.where(logits == m1, lane, N_EXPERTS), axis=1,
                 keepdims=True)
    masked = jnp.where(lane == i1, -jnp.inf, logits)
    m2 = jnp.max(masked, axis=1, keepdims=True)
    i2 = jnp.min(jnp.where(masked == m2, lane, N_EXPERTS), axis=1,
                 keepdims=True)
    w1 = jax.nn.sigmoid(m1 - m2)
    w2 = 1.0 - w1

    oh1 = (lane == i1).astype(jnp.float32)
    oh2 = (lane == i2).astype(jnp.float32)
    oh = oh1 + oh2

    r = jax.lax.broadcasted_iota(jnp.int32, (T_TOKENS, T_TOKENS), 0)
    c = jax.lax.broadcasted_iota(jnp.int32, (T_TOKENS, T_TOKENS), 1)
    tril = (c < r).astype(jnp.float32)
    cnt_before = jax.lax.dot_general(
        tril, oh, (((1,), (0,)), ((), ())),
        preferred_element_type=jnp.float32)
    counts = jnp.sum(oh, axis=0, keepdims=True)

    aligned = jnp.ceil(counts / BM) * BM
    e_r = jax.lax.broadcasted_iota(jnp.int32, (N_EXPERTS, N_EXPERTS), 0)
    e_c = jax.lax.broadcasted_iota(jnp.int32, (N_EXPERTS, N_EXPERTS), 1)
    tril8 = (e_r < e_c).astype(jnp.float32)
    start = jax.lax.dot_general(
        aligned, tril8, (((1,), (0,)), ((), ())),
        preferred_element_type=jnp.float32)

    rank1 = jnp.sum(oh1 * cnt_before, axis=1, keepdims=True)
    rank2 = jnp.sum(oh2 * cnt_before, axis=1, keepdims=True)
    base1 = jnp.sum(oh1 * start, axis=1, keepdims=True)
    base2 = jnp.sum(oh2 * start, axis=1, keepdims=True)
    slot1 = (base1 + rank1).astype(jnp.int32)
    slot2 = (base2 + rank2).astype(jnp.int32)

    slots_ref[...] = jnp.concatenate([slot1, slot2], axis=1)
    wts_ref[...] = jnp.concatenate([w1, w2], axis=1)
    counts_ref[...] = counts


def _run_router(x, gate_w, gate_b):
    return pl.pallas_call(
        _router_body,
        out_shape=(
            jax.ShapeDtypeStruct((T_TOKENS, 2), jnp.int32),
            jax.ShapeDtypeStruct((T_TOKENS, 2), jnp.float32),
            jax.ShapeDtypeStruct((1, N_EXPERTS), jnp.float32),
        ),
    )(x, gate_w, gate_b.reshape(1, N_EXPERTS))


def _make_dispatch():
    mesh = plsc.VectorSubcoreMesh(core_axis_name="c", subcore_axis_name="s")

    @functools.partial(
        pl.kernel, mesh=mesh,
        out_type=jax.ShapeDtypeStruct((NPAD, D_MODEL), jnp.float32),
        scratch_types=[
            pltpu.VMEM((2, TPW), jnp.int32),
            pltpu.VMEM((TPW, D_MODEL), jnp.float32),
            pltpu.SemaphoreType.DMA,
        ],
    )
    def dispatch(x_hbm, slots_hbm, xs_hbm, idx_v, rows_v, sem):
        wid = lax.axis_index("s") * 2 + lax.axis_index("c")
        base = wid * TPW
        pltpu.sync_copy(slots_hbm.at[wid], idx_v)
        pltpu.sync_copy(x_hbm.at[pl.ds(base, TPW)], rows_v)
        c0 = pltpu.async_copy(rows_v, xs_hbm.at[idx_v.at[0]], sem)
        c1 = pltpu.async_copy(rows_v, xs_hbm.at[idx_v.at[1]], sem)
        c0.wait()
        c1.wait()

    return dispatch


def _grouped_body(be_ref, act_ref, xs_ref, wglu_ref, wlin_ref, bglu_ref,
                  blin_ref, wd_ref, bd_ref, ys_ref):
    b = pl.program_id(0)

    @pl.when(act_ref[b] == 1)
    def _():
        xb = xs_ref[...]
        h_glu = jax.lax.dot_general(
            xb, wglu_ref[...], (((1,), (0,)), ((), ())),
            preferred_element_type=jnp.float32) + bglu_ref[...]
        h_lin = jax.lax.dot_general(
            xb, wlin_ref[...], (((1,), (0,)), ((), ())),
            preferred_element_type=jnp.float32) + blin_ref[...]
        h_glu = jnp.minimum(h_glu, LIMIT)
        h_lin = jnp.clip(h_lin, -LIMIT, LIMIT)
        act = h_glu * jax.nn.sigmoid(ALPHA * h_glu) * (h_lin + BETA)
        y = jax.lax.dot_general(
            act, wd_ref[...], (((1,), (0,)), ((), ())),
            preferred_element_type=jnp.float32)
        ys_ref[...] = y + bd_ref[...]


def _run_grouped(xs, w_gate_up, bgu3, w_down, bd3, block_expert, active):
    grid_spec = pltpu.PrefetchScalarGridSpec(
        num_scalar_prefetch=2,
        grid=(NBLK,),
        in_specs=[
            pl.BlockSpec((BM, D_MODEL), lambda b, be, act: (b, 0)),
            pl.BlockSpec((None, D_MODEL, D_FF),
                         lambda b, be, act: (be[b], 0, 0)),
            pl.BlockSpec((None, D_MODEL, D_FF),
                         lambda b, be, act: (be[b], 0, 1)),
            pl.BlockSpec((None, 1, D_FF), lambda b, be, act: (be[b], 0, 0)),
            pl.BlockSpec((None, 1, D_FF), lambda b, be, act: (be[b], 0, 1)),
            pl.BlockSpec((None, D_FF, D_MODEL),
                         lambda b, be, act: (be[b], 0, 0)),
            pl.BlockSpec((None, 1, D_MODEL),
                         lambda b, be, act: (be[b], 0, 0)),
        ],
        out_specs=pl.BlockSpec((BM, D_MODEL), lambda b, be, act: (b, 0)),
    )
    return pl.pallas_call(
        _grouped_body,
        grid_spec=grid_spec,
        out_shape=jax.ShapeDtypeStruct((NPAD, D_MODEL), jnp.float32),
    )(block_expert, active, xs, w_gate_up, w_gate_up, bgu3, bgu3,
      w_down, bd3)


def _make_combine():
    mesh = plsc.VectorSubcoreMesh(core_axis_name="c", subcore_axis_name="s")

    nch = TPW // CH

    @functools.partial(
        pl.kernel, mesh=mesh,
        out_type=jax.ShapeDtypeStruct((T_TOKENS, D_MODEL), jnp.float32),
        scratch_types=[
            pltpu.VMEM((2, TPW), jnp.int32),
            pltpu.VMEM((2, TPW, 16), jnp.float32),
            pltpu.VMEM((2, CH, D_MODEL), jnp.float32),
            pltpu.VMEM((2, CH, D_MODEL), jnp.float32),
            pltpu.VMEM((2, CH, D_MODEL), jnp.float32),
            pltpu.SemaphoreType.DMA,
            pltpu.SemaphoreType.DMA,
            pltpu.SemaphoreType.DMA,
        ],
    )
    def combine(ys_hbm, slots_hbm, wts_hbm, out_hbm, idx_v, w_v, r0, r1, o,
                sem_a, sem_b, sem_w):
        wid = lax.axis_index("s") * 2 + lax.axis_index("c")
        base = wid * TPW
        pltpu.sync_copy(slots_hbm.at[wid], idx_v)
        pltpu.sync_copy(wts_hbm.at[wid], w_v)
        gsems = [sem_a, sem_b]

        def start_gathers(ci):
            bi = ci % 2
            g0 = pltpu.async_copy(
                ys_hbm.at[idx_v.at[0, pl.ds(ci * CH, CH)]], r0.at[bi],
                gsems[bi])
            g1 = pltpu.async_copy(
                ys_hbm.at[idx_v.at[1, pl.ds(ci * CH, CH)]], r1.at[bi],
                gsems[bi])
            return g0, g1

        pend = start_gathers(0)
        writes = []
        for ci in range(nch):
            bi = ci % 2
            g0, g1 = pend
            if ci + 1 < nch:
                nxt = start_gathers(ci + 1)
            g0.wait()
            g1.wait()
            if ci >= 2:
                writes[ci - 2].wait()
            for j in range(CH):
                w0 = w_v[0, ci * CH + j, :]
                w1 = w_v[1, ci * CH + j, :]

                def chunk_add(k, _, bi=bi, j=j, w0=w0, w1=w1):
                    sl = pl.ds(k * 16, 16)
                    o[bi, j, sl] = (w0 * r0[bi, j, sl]
                                    + w1 * r1[bi, j, sl])
                    return 0
                lax.fori_loop(0, D_MODEL // 16, chunk_add, 0)
            writes.append(pltpu.async_copy(
                o.at[bi], out_hbm.at[pl.ds(base + ci * CH, CH)], sem_w))
            if ci + 1 < nch:
                pend = nxt
        writes[-2].wait()
        writes[-1].wait()

    return combine


@jax.jit
def kernel(x, gate_w, gate_b, w_gate_up, b_gate_up, w_down, b_down):
    slots, wts, counts = _run_router(x, gate_w, gate_b)

    nb = jnp.ceil(counts[0] / BM).astype(jnp.int32)
    cnb = jnp.cumsum(nb)
    total_nb = cnb[-1]
    barange = jnp.arange(NBLK, dtype=jnp.int32)
    be_raw = jnp.minimum(
        (barange[:, None] >= cnb[None, :]).sum(axis=1), N_EXPERTS - 1
    ).astype(jnp.int32)
    active = (barange < total_nb).astype(jnp.int32)
    last_e = be_raw[jnp.maximum(total_nb - 1, 0)]
    block_expert = jnp.where(active == 1, be_raw, last_e)

    slots_w = slots.reshape(NW, TPW, 2).transpose(0, 2, 1)
    wts_w = wts.reshape(NW, TPW, 2).transpose(0, 2, 1)
    wts_wb = jnp.broadcast_to(wts_w[..., None],
                              (NW, 2, TPW, 16)) + jnp.zeros(
        (NW, 2, TPW, 16), jnp.float32)

    xs = _make_dispatch()(x, slots_w)

    bgu3 = b_gate_up.reshape(N_EXPERTS, 1, 2 * D_FF)
    bd3 = b_down.reshape(N_EXPERTS, 1, D_MODEL)
    ys = _run_grouped(xs, w_gate_up, bgu3, w_down, bd3, block_expert, active)

    out = _make_combine()(ys, slots_w, wts_wb)
    return out
